# Initial kernel scaffold; baseline (speedup 1.0000x reference)
import jax
import jax.numpy as jnp
from jax.experimental import pallas as pl


def kernel(x, edge_index, W1, b1, W2, b2):
    raise NotImplementedError("write your pallas kernel here")


# speedup vs baseline: 13.7312x; 13.7312x over previous
import functools

import jax
import jax.numpy as jnp
from jax import lax
from jax.experimental import pallas as pl
from jax.experimental.pallas import tpu as pltpu
from jax.experimental.pallas import tpu_sc as plsc

N = 10000
NP = 10240
D = 128
L2W = 16
NC, NS, LANES = 2, 16, 16
CH = 128
STRIPE = NP // NS


def _sc_mesh():
    return plsc.VectorSubcoreMesh(
        core_axis_name="c", subcore_axis_name="s", num_cores=NC, num_subcores=NS
    )


def _fill(buf, nrows, ncols, val):
    def row(i, _):
        def col(j, _):
            buf[i, pl.ds(pl.multiple_of(j * LANES, LANES), LANES)] = jnp.full(
                (LANES,), val, jnp.float32)
            return 0
        return lax.fori_loop(0, ncols // LANES, col, 0)
    lax.fori_loop(0, nrows, row, 0)


def _make_sc_scatter(EP, W, gather):
    EPW = EP // (NC * NS)
    NCHUNK = EPW // CH

    def body(*refs):
        if gather:
            src_hbm, dst_hbm, tab_hbm, out_hbm, sidx_v, didx_v, rows_v, acc_sh, sem = refs
        else:
            dst_hbm, out_hbm, didx_v, rows_v, acc_sh, sem = refs
        c = lax.axis_index("c")
        s = lax.axis_index("s")
        wid = s * NC + c

        _fill(rows_v, CH, W, 0.0)
        def zs(i, _):
            off = pl.multiple_of(s * STRIPE + i * CH, CH)
            pltpu.sync_copy(rows_v, acc_sh.at[pl.ds(off, CH)])
            return 0
        lax.fori_loop(0, STRIPE // CH, zs, 0)
        if not gather:
            _fill(rows_v, CH, W, 1.0)
        plsc.subcore_barrier()

        base = wid * EPW
        def step(j, _):
            off = pl.multiple_of(base + j * CH, 8)
            pltpu.sync_copy(dst_hbm.at[pl.ds(off, CH)], didx_v)
            if gather:
                pltpu.sync_copy(src_hbm.at[pl.ds(off, CH)], sidx_v)
                pltpu.async_copy(tab_hbm.at[sidx_v], rows_v, sem).wait()
            pltpu.sync_copy(rows_v, acc_sh.at[didx_v], add=True)
            return 0
        lax.fori_loop(0, NCHUNK, step, 0)
        plsc.subcore_barrier()

        off = pl.multiple_of(s * STRIPE, 8)
        pltpu.sync_copy(acc_sh.at[pl.ds(off, STRIPE)],
                        out_hbm.at[c, pl.ds(off, STRIPE)])

    scratch = []
    if gather:
        scratch.append(pltpu.VMEM((CH,), jnp.int32))
    scratch += [
        pltpu.VMEM((CH,), jnp.int32),
        pltpu.VMEM((CH, W), jnp.float32),
        pltpu.VMEM_SHARED((NP, W), jnp.float32),
        pltpu.SemaphoreType.DMA,
    ]
    return pl.kernel(
        body,
        out_type=jax.ShapeDtypeStruct((NC, NP, W), jnp.float32),
        mesh=_sc_mesh(),
        scratch_types=scratch,
        compiler_params=pltpu.CompilerParams(use_tc_tiling_on_sc=False),
    )


def _tc1(x_p, W1, d0, d1):
    B = 2048
    def body(x_ref, w_ref, d0_ref, d1_ref, o_ref):
        dinv = lax.rsqrt(1.0 + d0_ref[:, 0:1] + d1_ref[:, 0:1])
        h = jnp.dot(x_ref[...], w_ref[...], preferred_element_type=jnp.float32)
        o_ref[...] = h * dinv
    return pl.pallas_call(
        body,
        grid=(NP // B,),
        in_specs=[
            pl.BlockSpec((B, D), lambda i: (i, 0)),
            pl.BlockSpec((D, D), lambda i: (0, 0)),
            pl.BlockSpec((B, L2W), lambda i: (i, 0)),
            pl.BlockSpec((B, L2W), lambda i: (i, 0)),
        ],
        out_specs=pl.BlockSpec((B, D), lambda i: (i, 0)),
        out_shape=jax.ShapeDtypeStruct((NP, D), jnp.float32),
    )(x_p, W1, d0, d1)


def _tc2(a0, a1, g1, d0, d1, b1r, w2p):
    B = 2048
    def body(a0_ref, a1_ref, g1_ref, d0_ref, d1_ref, b1_ref, w2_ref, o_ref):
        dinv = lax.rsqrt(1.0 + d0_ref[:, 0:1] + d1_ref[:, 0:1])
        pre = dinv * (a0_ref[...] + a1_ref[...] + g1_ref[...]) + b1_ref[...]
        h = jnp.maximum(pre, 0.0)
        o_ref[...] = jnp.dot(h, w2_ref[...],
                             preferred_element_type=jnp.float32) * dinv
    return pl.pallas_call(
        body,
        grid=(NP // B,),
        in_specs=[
            pl.BlockSpec((B, D), lambda i: (i, 0)),
            pl.BlockSpec((B, D), lambda i: (i, 0)),
            pl.BlockSpec((B, D), lambda i: (i, 0)),
            pl.BlockSpec((B, L2W), lambda i: (i, 0)),
            pl.BlockSpec((B, L2W), lambda i: (i, 0)),
            pl.BlockSpec((1, D), lambda i: (0, 0)),
            pl.BlockSpec((D, L2W), lambda i: (0, 0)),
        ],
        out_specs=pl.BlockSpec((B, L2W), lambda i: (i, 0)),
        out_shape=jax.ShapeDtypeStruct((NP, L2W), jnp.float32),
    )(a0, a1, g1, d0, d1, b1r, w2p)


def _tc3(a0, a1, g2, d0, d1, b2r):
    B = 2000
    def body(a0_ref, a1_ref, g2_ref, d0_ref, d1_ref, b2_ref, o_ref):
        dinv = lax.rsqrt(1.0 + d0_ref[:, 0:1] + d1_ref[:, 0:1])
        pre = dinv * (a0_ref[...] + a1_ref[...] + g2_ref[...]) + b2_ref[...]
        logit = pre[:, 0:2]
        m = jnp.max(logit, axis=1, keepdims=True)
        e = jnp.exp(logit - m)
        o_ref[...] = e / jnp.sum(e, axis=1, keepdims=True)
    return pl.pallas_call(
        body,
        grid=(N // B,),
        in_specs=[
            pl.BlockSpec((B, L2W), lambda i: (i, 0)),
            pl.BlockSpec((B, L2W), lambda i: (i, 0)),
            pl.BlockSpec((B, L2W), lambda i: (i, 0)),
            pl.BlockSpec((B, L2W), lambda i: (i, 0)),
            pl.BlockSpec((B, L2W), lambda i: (i, 0)),
            pl.BlockSpec((1, L2W), lambda i: (0, 0)),
        ],
        out_specs=pl.BlockSpec((B, 2), lambda i: (i, 0)),
        out_shape=jax.ShapeDtypeStruct((N, 2), jnp.float32),
    )(a0, a1, g2, d0, d1, b2r)


def kernel(x, edge_index, W1, b1, W2, b2):
    E = edge_index.shape[1]
    EP = ((E + NC * NS * CH - 1) // (NC * NS * CH)) * (NC * NS * CH)
    pad = EP - E

    src = edge_index[0].astype(jnp.int32)
    dst = edge_index[1].astype(jnp.int32)
    if pad:
        fill = jnp.full((pad,), N, jnp.int32)
        src = jnp.concatenate([src, fill])
        dst = jnp.concatenate([dst, fill])
    x_p = jnp.pad(x, ((0, NP - N), (0, 0)))
    b1r = b1[None, :]
    w2p = jnp.pad(W2, ((0, 0), (0, L2W - W2.shape[1])))
    b2r = jnp.pad(b2, (0, L2W - b2.shape[0]))[None, :]

    degp = _make_sc_scatter(EP, L2W, gather=False)(dst)
    d0, d1 = degp[0], degp[1]
    g1 = _tc1(x_p, W1, d0, d1)
    acc1 = _make_sc_scatter(EP, D, gather=True)(src, dst, g1)
    g2 = _tc2(acc1[0], acc1[1], g1, d0, d1, b1r, w2p)
    acc2 = _make_sc_scatter(EP, L2W, gather=True)(src, dst, g2)
    return _tc3(acc2[0], acc2[1], g2, d0, d1, b2r)
